# Initial kernel scaffold; baseline (speedup 1.0000x reference)
import jax
import jax.numpy as jnp
from jax.experimental import pallas as pl


def kernel(seg, embedding, offset_pred, z_pred, intrinsic, extrinsic):
    raise NotImplementedError("write your pallas kernel here")


# speedup vs baseline: 1126.5556x; 1126.5556x over previous
import functools
import jax
import jax.numpy as jnp
from jax import lax
from jax.experimental import pallas as pl
from jax.experimental.pallas import tpu as pltpu
from jax.experimental.pallas import tpu_sc as plsc

POST_CONF = 0.97
MARGIN2 = 25.0
MIN_CLUSTER = 100
B, ND, H, W = 4, 8, 144, 256
KMAX = H * W
MAX_LANES = 4
ACAP = 4096
CCAP = 1024
SLOTS = 48
GCH = 512
NC = 2
INF = float("inf")

_mesh = plsc.VectorSubcoreMesh(core_axis_name="c", subcore_axis_name="s")


def _iota16():
    return lax.iota(jnp.int32, 16)


@functools.partial(
    pl.kernel,
    mesh=_mesh,
    compiler_params=pltpu.CompilerParams(needs_layout_passes=False),
    out_type=[
        jax.ShapeDtypeStruct((B, ACAP), jnp.int32),
        jax.ShapeDtypeStruct((B, ACAP), jnp.int32),
        jax.ShapeDtypeStruct((B, CCAP), jnp.int32),
        jax.ShapeDtypeStruct((B, 32), jnp.int32),
    ],
    scratch_types=[
        pltpu.VMEM((KMAX,), jnp.float32),
        pltpu.VMEM((ACAP,), jnp.int32),
        pltpu.VMEM((ND * ACAP,), jnp.float32),
        pltpu.VMEM((ND * CCAP,), jnp.float32),
        pltpu.VMEM((CCAP,), jnp.int32),
        pltpu.VMEM((ACAP,), jnp.int32),
        pltpu.VMEM((32,), jnp.int32),
    ],
)
def _cluster_kernel(seg_hbm, emb_hbm,
                    aidx_out, cids_out, counts_out, meta_out,
                    segbuf, aidx, aemb, centers, counts, cids, meta):
    wid = lax.axis_index("s") * NC + lax.axis_index("c")

    @pl.when(wid < B)
    def _():
        b = wid
        zi16 = jnp.zeros((16,), jnp.int32)

        pltpu.sync_copy(seg_hbm.at[b], segbuf)

        def zero_a(i, _):
            aidx[pl.ds(i * 16, 16)] = zi16
            cids[pl.ds(i * 16, 16)] = zi16
            return 0
        lax.fori_loop(0, ACAP // 16, zero_a, 0)

        def zero_c(i, _):
            counts[pl.ds(i * 16, 16)] = zi16
            return 0
        lax.fori_loop(0, CCAP // 16, zero_c, 0)

        def comp_body(c, off):
            sv = segbuf[pl.ds(c * 16, 16)]
            m = sv >= POST_CONF
            idxv = _iota16() + c * 16
            mi = m.astype(jnp.int32)
            incl = plsc.cumsum(mi)
            pos = jnp.minimum(off + (incl - mi), ACAP - 1)
            plsc.store_scatter(aidx, [pos], idxv, mask=m)
            return off + jnp.sum(mi)
        off_fin = lax.fori_loop(0, KMAX // 16, comp_body, jnp.int32(0))
        n = jnp.minimum(off_fin, ACAP)

        nchp = (n + 15) // 16
        for d in range(ND):
            pltpu.sync_copy(emb_hbm.at[b * ND + d], segbuf)

            def emb_body(c, _, d=d):
                pidx = aidx[pl.ds(c * 16, 16)]
                aemb[pl.ds(d * ACAP + c * 16, 16)] = plsc.load_gather(
                    segbuf, [pidx])
                return 0
            lax.fori_loop(0, nchp, emb_body, 0)

        lanes = _iota16()
        m_lo8 = lanes < 8
        m_l0 = lanes == 0
        ones_i = jnp.ones((16,), jnp.int32)

        def point_body(i, ncent):
            iv = jnp.full((16,), i, jnp.int32)
            ev = plsc.load_gather(aemb, [lanes * ACAP + iv], mask=m_lo8)
            bc = [plsc.load_gather(aemb, [jnp.full((16,), d * ACAP, jnp.int32) + iv])
                  for d in range(ND)]

            nch = (ncent + 15) // 16

            def ch_body(c, carry):
                bd, bi = carry
                base = c * 16
                qs = []
                for d in range(ND):
                    cd = centers[pl.ds(d * CCAP + base, 16)]
                    dd = cd - bc[d]
                    qs.append(dd * dd)
                acc = ((qs[0] + qs[1]) + (qs[2] + qs[3])) + (
                    (qs[4] + qs[5]) + (qs[6] + qs[7]))
                lid = _iota16() + base
                dm = jnp.where(lid < ncent, acc, INF)
                better = dm < bd
                return jnp.where(better, dm, bd), jnp.where(better, lid, bi)

            bd0 = jnp.full((16,), INF)
            bi0 = jnp.full((16,), KMAX, jnp.int32)
            bd, bi = lax.fori_loop(0, nch, ch_body, (bd0, bi0))

            d2min = jnp.min(bd)
            idxmin = jnp.min(jnp.where(bd == d2min, bi, KMAX))
            merge = (ncent > 0) & (d2min < MARGIN2)
            target = jnp.minimum(jnp.where(merge, idxmin, ncent), CCAP - 1)

            tv = jnp.full((16,), target, jnp.int32)
            cntf = plsc.load_gather(counts, [tv]).astype(jnp.float32)
            cidx = lanes * CCAP + tv
            cold = plsc.load_gather(centers, [cidx], mask=m_lo8)
            cnew = (cold * cntf + ev) / (cntf + 1.0)
            plsc.store_scatter(centers, [cidx], cnew, mask=m_lo8)
            plsc.addupdate_scatter(counts, [tv], ones_i, mask=m_l0)
            plsc.store_scatter(cids, [iv], tv, mask=m_l0)
            return ncent + jnp.where(merge, 0, 1)

        ncent = lax.fori_loop(0, n, point_body, jnp.int32(0))

        meta[pl.ds(0, 16)] = jnp.full((16,), n, jnp.int32)
        meta[pl.ds(16, 16)] = jnp.full((16,), ncent, jnp.int32)

        pltpu.sync_copy(aidx, aidx_out.at[b])
        pltpu.sync_copy(cids, cids_out.at[b])
        pltpu.sync_copy(counts, counts_out.at[b])
        pltpu.sync_copy(meta, meta_out.at[b])


@functools.partial(
    pl.kernel,
    mesh=_mesh,
    compiler_params=pltpu.CompilerParams(needs_layout_passes=False),
    out_type=[
        jax.ShapeDtypeStruct((B, MAX_LANES * H), jnp.float32),
        jax.ShapeDtypeStruct((B, MAX_LANES * H), jnp.float32),
        jax.ShapeDtypeStruct((B, MAX_LANES * H), jnp.float32),
        jax.ShapeDtypeStruct((B, 16), jnp.int32),
    ],
    scratch_types=[
        pltpu.VMEM((KMAX,), jnp.float32),
        pltpu.VMEM((ACAP,), jnp.int32),
        pltpu.VMEM((ACAP,), jnp.int32),
        pltpu.VMEM((CCAP,), jnp.int32),
        pltpu.VMEM((32,), jnp.int32),
        pltpu.VMEM((ACAP,), jnp.float32),
        pltpu.VMEM((ACAP,), jnp.float32),
        pltpu.VMEM((CCAP,), jnp.int32),
        pltpu.VMEM((SLOTS * H,), jnp.int32),
        pltpu.VMEM((SLOTS * H,), jnp.float32),
        pltpu.VMEM((SLOTS * H,), jnp.float32),
        pltpu.VMEM((MAX_LANES * H,), jnp.float32),
        pltpu.VMEM((MAX_LANES * H,), jnp.float32),
        pltpu.VMEM((MAX_LANES * H,), jnp.float32),
        pltpu.VMEM((16,), jnp.int32),
    ],
)
def _lanes_kernel(off_hbm, z_hbm, aidx_hbm, cids_hbm, counts_hbm, meta_hbm,
                  ox_out, oy_out, oz_out, cnt_out,
                  imgbuf, aidx, cids, counts, meta, xadj, zact, slotmap,
                  scnt, ssx, ssz, obx, oby, obz, cntv):
    wid = lax.axis_index("s") * NC + lax.axis_index("c")

    @pl.when(wid < B)
    def _():
        b = wid
        lanes = _iota16()

        pltpu.sync_copy(meta_hbm.at[b], meta)
        pltpu.sync_copy(aidx_hbm.at[b], aidx)
        pltpu.sync_copy(cids_hbm.at[b], cids)
        pltpu.sync_copy(counts_hbm.at[b], counts)
        n = jnp.max(meta[pl.ds(0, 16)])
        nchp = (n + 15) // 16

        pltpu.sync_copy(off_hbm.at[b], imgbuf)

        def xa_body(c, _):
            pidx = aidx[pl.ds(c * 16, 16)]
            ov = plsc.load_gather(imgbuf, [pidx])
            sig = 1.0 / (1.0 + jnp.exp(-ov))
            col = (pidx & (W - 1)).astype(jnp.float32)
            xadj[pl.ds(c * 16, 16)] = col + sig
            return 0
        lax.fori_loop(0, nchp, xa_body, 0)

        pltpu.sync_copy(z_hbm.at[b], imgbuf)

        def z_body(c, _):
            pidx = aidx[pl.ds(c * 16, 16)]
            zact[pl.ds(c * 16, 16)] = plsc.load_gather(imgbuf, [pidx])
            return 0
        lax.fori_loop(0, nchp, z_body, 0)

        def slot_body(c, s):
            cv = counts[pl.ds(c * 16, 16)]
            el = cv >= MIN_CLUSTER
            eli = el.astype(jnp.int32)
            incl = plsc.cumsum(eli)
            slotv = jnp.where(el, s + (incl - eli), -1)
            slotv = jnp.where(slotv < SLOTS, slotv, -1)
            slotmap[pl.ds(c * 16, 16)] = slotv
            return s + jnp.sum(eli)
        lax.fori_loop(0, CCAP // 16, slot_body, jnp.int32(0))

        zf16 = jnp.zeros((16,), jnp.float32)
        zi16 = jnp.zeros((16,), jnp.int32)

        def zs_body(i, _):
            scnt[pl.ds(i * 16, 16)] = zi16
            ssx[pl.ds(i * 16, 16)] = zf16
            ssz[pl.ds(i * 16, 16)] = zf16
            return 0
        lax.fori_loop(0, SLOTS * H // 16, zs_body, 0)

        def zo_body(i, _):
            obx[pl.ds(i * 16, 16)] = zf16
            oby[pl.ds(i * 16, 16)] = zf16
            obz[pl.ds(i * 16, 16)] = zf16
            return 0
        lax.fori_loop(0, MAX_LANES * H // 16, zo_body, 0)
        cntv[pl.ds(0, 16)] = zi16

        ones_i = jnp.ones((16,), jnp.int32)

        def st_body(c, _):
            base = c * 16
            pidx = aidx[pl.ds(base, 16)]
            cid = cids[pl.ds(base, 16)]
            slot = plsc.load_gather(slotmap, [cid])
            valid = (slot >= 0) & ((lanes + base) < n)
            rows = pidx >> 8
            sidx = jnp.where(valid, slot * H + rows, 0)
            xv = xadj[pl.ds(base, 16)]
            zv = zact[pl.ds(base, 16)]
            for j in range(16):
                mj = (lanes == j) & valid
                plsc.addupdate_scatter(scnt, [sidx], ones_i, mask=mj)
                plsc.addupdate_scatter(ssx, [sidx], xv, mask=mj)
                plsc.addupdate_scatter(ssz, [sidx], zv, mask=mj)
            return 0
        lax.fori_loop(0, nchp, st_body, 0)

        def lane_body(s, lane):
            def nr_body(c9, acc):
                cv = scnt[pl.ds(s * H + c9 * 16, 16)]
                return acc + jnp.sum((cv > 0).astype(jnp.int32))
            nr = lax.fori_loop(0, H // 16, nr_body, jnp.int32(0))
            cand = nr >= 2
            do_lane = cand & (lane < MAX_LANES)

            @pl.when(do_lane)
            def _():
                def row_body(t, pos_carry):
                    c9 = (H // 16 - 1) - t
                    o = s * H + c9 * 16
                    cv = scnt[pl.ds(o, 16)]
                    pres = cv > 0
                    pi = pres.astype(jnp.int32)
                    denom = jnp.where(pres, cv, 1).astype(jnp.float32)
                    mean_x = ssx[pl.ds(o, 16)] / denom
                    mean_z = ssz[pl.ds(o, 16)] / denom
                    rr = (lanes + c9 * 16).astype(jnp.float32)
                    xvv = (100.0 - (rr + 0.5)) * 0.5
                    yvv = 64.0 - 0.5 * mean_x
                    incl = plsc.cumsum(lax.rev(pi, (0,)))
                    pos_local = lax.rev(incl, (0,)) - pi
                    pos = pos_local + pos_carry + lane * H
                    plsc.store_scatter(obx, [pos], xvv, mask=pres)
                    plsc.store_scatter(oby, [pos], yvv, mask=pres)
                    plsc.store_scatter(obz, [pos], mean_z, mask=pres)
                    return pos_carry + jnp.sum(pi)
                n_l = lax.fori_loop(0, H // 16, row_body, jnp.int32(0))
                plsc.store_scatter(
                    cntv, [jnp.full((16,), lane, jnp.int32)],
                    jnp.full((16,), n_l, jnp.int32), mask=lanes == 0)

            return lane + jnp.where(cand, 1, 0)
        lax.fori_loop(0, SLOTS, lane_body, jnp.int32(0))

        pltpu.sync_copy(obx, ox_out.at[b])
        pltpu.sync_copy(oby, oy_out.at[b])
        pltpu.sync_copy(obz, oz_out.at[b])
        pltpu.sync_copy(cntv, cnt_out.at[b])


def kernel(seg, embedding, offset_pred, z_pred, intrinsic, extrinsic):
    segf = seg.reshape(B, KMAX)
    embf = embedding.reshape(B * ND, KMAX)
    offf = offset_pred.reshape(B, KMAX)
    zf = z_pred.reshape(B, KMAX)

    aidx, cids, counts, meta = _cluster_kernel(segf, embf)
    ox, oy, oz, cnt = _lanes_kernel(offf, zf, aidx, cids, counts, meta)

    out = jnp.stack(
        [ox.reshape(B, MAX_LANES, H),
         oy.reshape(B, MAX_LANES, H),
         oz.reshape(B, MAX_LANES, H)], axis=-1)
    return out, cnt[:, :MAX_LANES]


# speedup vs baseline: 1142.9733x; 1.0146x over previous
import functools
import jax
import jax.numpy as jnp
from jax import lax
from jax.experimental import pallas as pl
from jax.experimental.pallas import tpu as pltpu
from jax.experimental.pallas import tpu_sc as plsc

POST_CONF = 0.97
MARGIN2 = 25.0
MIN_CLUSTER = 100
B, ND, H, W = 4, 8, 144, 256
KMAX = H * W
MAX_LANES = 4
ACAP = 4096
CCAP = 1024
SLOTS = 48
NC = 2
INF = float("inf")

_mesh = plsc.VectorSubcoreMesh(core_axis_name="c", subcore_axis_name="s")

_GDN = lax.GatherDimensionNumbers(
    offset_dims=(), collapsed_slice_dims=(0,), start_index_map=(0,))


def _iota16():
    return lax.iota(jnp.int32, 16)


def _vbcast(vec, idx_vec):
    return lax.gather(vec, idx_vec[:, None], _GDN, (1,),
                      mode=lax.GatherScatterMode.PROMISE_IN_BOUNDS)


@functools.partial(
    pl.kernel,
    mesh=_mesh,
    compiler_params=pltpu.CompilerParams(needs_layout_passes=False),
    out_type=[
        jax.ShapeDtypeStruct((B, MAX_LANES * H), jnp.float32),
        jax.ShapeDtypeStruct((B, MAX_LANES * H), jnp.float32),
        jax.ShapeDtypeStruct((B, MAX_LANES * H), jnp.float32),
        jax.ShapeDtypeStruct((B, 16), jnp.int32),
    ],
    scratch_types=[
        pltpu.VMEM((KMAX,), jnp.float32),
        pltpu.VMEM((ACAP,), jnp.int32),
        pltpu.VMEM((ND * ACAP,), jnp.float32),
        pltpu.VMEM((ND * CCAP,), jnp.float32),
        pltpu.VMEM((CCAP,), jnp.int32),
        pltpu.VMEM((ACAP,), jnp.int32),
        pltpu.VMEM((ACAP,), jnp.float32),
        pltpu.VMEM((ACAP,), jnp.float32),
        pltpu.VMEM((CCAP,), jnp.int32),
        pltpu.VMEM((SLOTS * H,), jnp.int32),
        pltpu.VMEM((SLOTS * H,), jnp.float32),
        pltpu.VMEM((SLOTS * H,), jnp.float32),
        pltpu.VMEM((MAX_LANES * H,), jnp.float32),
        pltpu.VMEM((MAX_LANES * H,), jnp.float32),
        pltpu.VMEM((MAX_LANES * H,), jnp.float32),
        pltpu.VMEM((16,), jnp.int32),
    ],
)
def _fused_kernel(seg_hbm, emb_hbm, off_hbm, z_hbm,
                 ox_out, oy_out, oz_out, cnt_out,
                 imgbuf, aidx, aemb, centers, counts, cids, xadj, zact,
                 slotmap, scnt, ssx, ssz, obx, oby, obz, cntv):
    wid = lax.axis_index("s") * NC + lax.axis_index("c")

    @pl.when(wid < B)
    def _():
        b = wid
        lanes = _iota16()
        zi16 = jnp.zeros((16,), jnp.int32)
        zf16 = jnp.zeros((16,), jnp.float32)

        pltpu.sync_copy(seg_hbm.at[b], imgbuf)

        def zero_a(i, _):
            aidx[pl.ds(i * 16, 16)] = zi16
            cids[pl.ds(i * 16, 16)] = zi16
            return 0
        lax.fori_loop(0, ACAP // 16, zero_a, 0)

        def zero_c(i, _):
            counts[pl.ds(i * 16, 16)] = zi16
            return 0
        lax.fori_loop(0, CCAP // 16, zero_c, 0)

        if True:
            l15 = jnp.full((16,), 15, jnp.int32)

            def comp_body(c, offv):
                sv = imgbuf[pl.ds(c * 16, 16)]
                m = sv >= POST_CONF
                idxv = _iota16() + c * 16
                mi = m.astype(jnp.int32)
                incl = plsc.cumsum(mi)
                pos = jnp.minimum(offv + (incl - mi), ACAP - 1)
                plsc.store_scatter(aidx, [pos], idxv, mask=m)
                return offv + _vbcast(incl, l15)
            offv = lax.fori_loop(0, KMAX // 16, comp_body, zi16)
            n = jnp.minimum(jnp.max(offv), ACAP)

        nchp = (n + 15) // 16

        for d in range(ND):
            pltpu.sync_copy(emb_hbm.at[b * ND + d], imgbuf)

            def emb_body(c, _, d=d):
                pidx = aidx[pl.ds(c * 16, 16)]
                aemb[pl.ds(d * ACAP + c * 16, 16)] = plsc.load_gather(
                    imgbuf, [pidx])
                return 0
            lax.fori_loop(0, nchp, emb_body, 0)

        pltpu.sync_copy(off_hbm.at[b], imgbuf)

        def xa_body(c, _):
            pidx = aidx[pl.ds(c * 16, 16)]
            ov = plsc.load_gather(imgbuf, [pidx])
            sig = 1.0 / (1.0 + jnp.exp(-ov))
            col = (pidx & (W - 1)).astype(jnp.float32)
            xadj[pl.ds(c * 16, 16)] = col + sig
            return 0
        lax.fori_loop(0, nchp, xa_body, 0)

        pltpu.sync_copy(z_hbm.at[b], imgbuf)

        def z_body(c, _):
            pidx = aidx[pl.ds(c * 16, 16)]
            zact[pl.ds(c * 16, 16)] = plsc.load_gather(imgbuf, [pidx])
            return 0
        lax.fori_loop(0, nchp, z_body, 0)

        m_lo8 = lanes < 8
        m_l0 = lanes == 0
        ones_i = jnp.ones((16,), jnp.int32)

        def point_body(i, ncent):
            iv = jnp.full((16,), i, jnp.int32)
            ev = plsc.load_gather(aemb, [lanes * ACAP + iv], mask=m_lo8)
            bc = [plsc.load_gather(aemb,
                                   [jnp.full((16,), d * ACAP, jnp.int32) + iv])
                  for d in range(ND)]

            nch = (ncent + 15) // 16

            def ch_body(c, carry):
                bd, bi = carry
                base = c * 16
                qs = []
                for d in range(ND):
                    cd = centers[pl.ds(d * CCAP + base, 16)]
                    dd = cd - bc[d]
                    qs.append(dd * dd)
                acc = ((qs[0] + qs[1]) + (qs[2] + qs[3])) + (
                    (qs[4] + qs[5]) + (qs[6] + qs[7]))
                lid = _iota16() + base
                dm = jnp.where(lid < ncent, acc, INF)
                better = dm < bd
                return jnp.where(better, dm, bd), jnp.where(better, lid, bi)

            bd0 = jnp.full((16,), INF)
            bi0 = jnp.full((16,), KMAX, jnp.int32)
            bd, bi = lax.fori_loop(0, nch, ch_body, (bd0, bi0))

            d2min = jnp.min(bd)
            merge = (ncent > 0) & (d2min < MARGIN2)
            eqm = bd == d2min
            def _fast():
                ffs = plsc.all_reduce_ffs(eqm)
                return _vbcast(bi, jnp.minimum(ffs, 15))

            def _slow():
                return jnp.full((16,), jnp.min(jnp.where(eqm, bi, KMAX)),
                                jnp.int32)
            idxv = lax.cond(nch <= 1, _fast, _slow)

            tv = jnp.where(merge, idxv, jnp.full((16,), ncent, jnp.int32))
            tv = jnp.minimum(tv, CCAP - 1)
            cntf = plsc.load_gather(counts, [tv]).astype(jnp.float32)
            cidx = lanes * CCAP + tv
            cold = plsc.load_gather(centers, [cidx], mask=m_lo8)
            cnew = (cold * cntf + ev) / (cntf + 1.0)
            plsc.store_scatter(centers, [cidx], cnew, mask=m_lo8)
            plsc.addupdate_scatter(counts, [tv], ones_i, mask=m_l0)
            plsc.store_scatter(cids, [iv], tv, mask=m_l0)
            return ncent + jnp.where(merge, 0, 1)

        lax.fori_loop(0, n, point_body, jnp.int32(0))

        def slot_body(c, s):
            cv = counts[pl.ds(c * 16, 16)]
            el = cv >= MIN_CLUSTER
            eli = el.astype(jnp.int32)
            incl = plsc.cumsum(eli)
            slotv = jnp.where(el, s + (incl - eli), -1)
            slotv = jnp.where(slotv < SLOTS, slotv, -1)
            slotmap[pl.ds(c * 16, 16)] = slotv
            return s + jnp.sum(eli)
        lax.fori_loop(0, CCAP // 16, slot_body, jnp.int32(0))

        def zs_body(i, _):
            scnt[pl.ds(i * 16, 16)] = zi16
            ssx[pl.ds(i * 16, 16)] = zf16
            ssz[pl.ds(i * 16, 16)] = zf16
            return 0
        lax.fori_loop(0, SLOTS * H // 16, zs_body, 0)

        def zo_body(i, _):
            obx[pl.ds(i * 16, 16)] = zf16
            oby[pl.ds(i * 16, 16)] = zf16
            obz[pl.ds(i * 16, 16)] = zf16
            return 0
        lax.fori_loop(0, MAX_LANES * H // 16, zo_body, 0)
        cntv[pl.ds(0, 16)] = zi16

        def st_body(c, _):
            base = c * 16
            pidx = aidx[pl.ds(base, 16)]
            cid = cids[pl.ds(base, 16)]
            slot = plsc.load_gather(slotmap, [cid])
            valid = (slot >= 0) & ((lanes + base) < n)
            rows = pidx >> 8
            sidx = jnp.where(valid, slot * H + rows, 0)
            xv = xadj[pl.ds(base, 16)]
            zv = zact[pl.ds(base, 16)]
            for j in range(16):
                mj = (lanes == j) & valid
                plsc.addupdate_scatter(scnt, [sidx], ones_i, mask=mj)
                plsc.addupdate_scatter(ssx, [sidx], xv, mask=mj)
                plsc.addupdate_scatter(ssz, [sidx], zv, mask=mj)
            return 0
        lax.fori_loop(0, nchp, st_body, 0)

        def lane_body(s, lane):
            def nr_body(c9, acc):
                cv = scnt[pl.ds(s * H + c9 * 16, 16)]
                return acc + jnp.sum((cv > 0).astype(jnp.int32))
            nr = lax.fori_loop(0, H // 16, nr_body, jnp.int32(0))
            cand = nr >= 2
            do_lane = cand & (lane < MAX_LANES)

            @pl.when(do_lane)
            def _():
                def row_body(t, pos_carry):
                    c9 = (H // 16 - 1) - t
                    o = s * H + c9 * 16
                    cv = scnt[pl.ds(o, 16)]
                    pres = cv > 0
                    pi = pres.astype(jnp.int32)
                    denom = jnp.where(pres, cv, 1).astype(jnp.float32)
                    mean_x = ssx[pl.ds(o, 16)] / denom
                    mean_z = ssz[pl.ds(o, 16)] / denom
                    rr = (lanes + c9 * 16).astype(jnp.float32)
                    xvv = (100.0 - (rr + 0.5)) * 0.5
                    yvv = 64.0 - 0.5 * mean_x
                    incl = plsc.cumsum(lax.rev(pi, (0,)))
                    pos_local = lax.rev(incl, (0,)) - pi
                    pos = pos_local + pos_carry + lane * H
                    plsc.store_scatter(obx, [pos], xvv, mask=pres)
                    plsc.store_scatter(oby, [pos], yvv, mask=pres)
                    plsc.store_scatter(obz, [pos], mean_z, mask=pres)
                    return pos_carry + jnp.sum(pi)
                n_l = lax.fori_loop(0, H // 16, row_body, jnp.int32(0))
                plsc.store_scatter(
                    cntv, [jnp.full((16,), lane, jnp.int32)],
                    jnp.full((16,), n_l, jnp.int32), mask=lanes == 0)

            return lane + jnp.where(cand, 1, 0)
        lax.fori_loop(0, SLOTS, lane_body, jnp.int32(0))

        pltpu.sync_copy(obx, ox_out.at[b])
        pltpu.sync_copy(oby, oy_out.at[b])
        pltpu.sync_copy(obz, oz_out.at[b])
        pltpu.sync_copy(cntv, cnt_out.at[b])


def kernel(seg, embedding, offset_pred, z_pred, intrinsic, extrinsic):
    segf = seg.reshape(B, KMAX)
    embf = embedding.reshape(B * ND, KMAX)
    offf = offset_pred.reshape(B, KMAX)
    zf = z_pred.reshape(B, KMAX)

    ox, oy, oz, cnt = _fused_kernel(segf, embf, offf, zf)

    out = jnp.stack(
        [ox.reshape(B, MAX_LANES, H),
         oy.reshape(B, MAX_LANES, H),
         oz.reshape(B, MAX_LANES, H)], axis=-1)
    return out, cnt[:, :MAX_LANES]


# speedup vs baseline: 1372.0007x; 1.2004x over previous
import functools
import jax
import jax.numpy as jnp
from jax import lax
from jax.experimental import pallas as pl
from jax.experimental.pallas import tpu as pltpu
from jax.experimental.pallas import tpu_sc as plsc

POST_CONF = 0.97
MARGIN2 = 25.0
MIN_CLUSTER = 100
B, ND, H, W = 4, 8, 144, 256
KMAX = H * W
MAX_LANES = 4
ACAP = 4096
CCAP = 1024
SLOTS = 48
NC = 2
INF = float("inf")

_mesh = plsc.VectorSubcoreMesh(core_axis_name="c", subcore_axis_name="s")

_GDN = lax.GatherDimensionNumbers(
    offset_dims=(), collapsed_slice_dims=(0,), start_index_map=(0,))


def _iota16():
    return lax.iota(jnp.int32, 16)


def _vbcast(vec, idx_vec):
    return lax.gather(vec, idx_vec[:, None], _GDN, (1,),
                      mode=lax.GatherScatterMode.PROMISE_IN_BOUNDS)


@functools.partial(
    pl.kernel,
    mesh=_mesh,
    compiler_params=pltpu.CompilerParams(needs_layout_passes=False),
    out_type=[
        jax.ShapeDtypeStruct((B, MAX_LANES * H), jnp.float32),
        jax.ShapeDtypeStruct((B, MAX_LANES * H), jnp.float32),
        jax.ShapeDtypeStruct((B, MAX_LANES * H), jnp.float32),
        jax.ShapeDtypeStruct((B, 16), jnp.int32),
    ],
    scratch_types=[
        pltpu.VMEM((KMAX,), jnp.float32),
        pltpu.VMEM((ACAP,), jnp.int32),
        pltpu.VMEM((ND * ACAP,), jnp.float32),
        pltpu.VMEM((ND * CCAP,), jnp.float32),
        pltpu.VMEM((CCAP,), jnp.int32),
        pltpu.VMEM((ACAP,), jnp.int32),
        pltpu.VMEM((ACAP,), jnp.float32),
        pltpu.VMEM((ACAP,), jnp.float32),
        pltpu.VMEM((CCAP,), jnp.int32),
        pltpu.VMEM((SLOTS * H,), jnp.int32),
        pltpu.VMEM((SLOTS * H,), jnp.float32),
        pltpu.VMEM((SLOTS * H,), jnp.float32),
        pltpu.VMEM((MAX_LANES * H,), jnp.float32),
        pltpu.VMEM((MAX_LANES * H,), jnp.float32),
        pltpu.VMEM((MAX_LANES * H,), jnp.float32),
        pltpu.VMEM((16,), jnp.int32),
    ],
)
def _fused_kernel(seg_hbm, emb_hbm, off_hbm, z_hbm,
                 ox_out, oy_out, oz_out, cnt_out,
                 imgbuf, aidx, aemb, centers, counts, cids, xadj, zact,
                 slotmap, scnt, ssx, ssz, obx, oby, obz, cntv):
    wid = lax.axis_index("s") * NC + lax.axis_index("c")

    @pl.when(wid < B)
    def _():
        b = wid
        lanes = _iota16()
        zi16 = jnp.zeros((16,), jnp.int32)
        zf16 = jnp.zeros((16,), jnp.float32)

        pltpu.sync_copy(seg_hbm.at[b], imgbuf)

        @plsc.parallel_loop(0, ACAP // 16, unroll=4)
        def _(i):
            aidx[pl.ds(i * 16, 16)] = zi16
            cids[pl.ds(i * 16, 16)] = zi16

        @plsc.parallel_loop(0, CCAP // 16, unroll=4)
        def _(i):
            counts[pl.ds(i * 16, 16)] = zi16

        l15 = jnp.full((16,), 15, jnp.int32)

        @plsc.parallel_loop(0, KMAX // 16, unroll=4, carry=zi16)
        def offv(c, offv):
            sv = imgbuf[pl.ds(c * 16, 16)]
            m = sv >= POST_CONF
            idxv = _iota16() + c * 16
            mi = m.astype(jnp.int32)
            incl = plsc.cumsum(mi)
            pos = jnp.minimum(offv + (incl - mi), ACAP - 1)
            plsc.store_scatter(aidx, [pos], idxv, mask=m)
            return offv + _vbcast(incl, l15)
        n = jnp.minimum(jnp.max(offv), ACAP)

        nchp = (n + 15) // 16

        for d in range(ND):
            pltpu.sync_copy(emb_hbm.at[b * ND + d], imgbuf)

            @plsc.parallel_loop(0, nchp, unroll=4)
            def _(c, d=d):
                pidx = aidx[pl.ds(c * 16, 16)]
                aemb[pl.ds(d * ACAP + c * 16, 16)] = plsc.load_gather(
                    imgbuf, [pidx])

        pltpu.sync_copy(off_hbm.at[b], imgbuf)

        @plsc.parallel_loop(0, nchp, unroll=4)
        def _(c):
            pidx = aidx[pl.ds(c * 16, 16)]
            ov = plsc.load_gather(imgbuf, [pidx])
            sig = 1.0 / (1.0 + jnp.exp(-ov))
            col = (pidx & (W - 1)).astype(jnp.float32)
            xadj[pl.ds(c * 16, 16)] = col + sig

        pltpu.sync_copy(z_hbm.at[b], imgbuf)

        @plsc.parallel_loop(0, nchp, unroll=4)
        def _(c):
            pidx = aidx[pl.ds(c * 16, 16)]
            zact[pl.ds(c * 16, 16)] = plsc.load_gather(imgbuf, [pidx])

        m_lo8 = lanes < 8
        m_l0 = lanes == 0
        ones_i = jnp.ones((16,), jnp.int32)

        def point_body(i, ncent):
            iv = jnp.full((16,), i, jnp.int32)
            ev = plsc.load_gather(aemb, [lanes * ACAP + iv], mask=m_lo8)
            bc = [plsc.load_gather(aemb,
                                   [jnp.full((16,), d * ACAP, jnp.int32) + iv])
                  for d in range(ND)]

            nch = (ncent + 15) // 16

            def ch_body(c, carry):
                bd, bi = carry
                base = c * 16
                qs = []
                for d in range(ND):
                    cd = centers[pl.ds(d * CCAP + base, 16)]
                    dd = cd - bc[d]
                    qs.append(dd * dd)
                acc = ((qs[0] + qs[1]) + (qs[2] + qs[3])) + (
                    (qs[4] + qs[5]) + (qs[6] + qs[7]))
                lid = _iota16() + base
                dm = jnp.where(lid < ncent, acc, INF)
                better = dm < bd
                return jnp.where(better, dm, bd), jnp.where(better, lid, bi)

            bd0 = jnp.full((16,), INF)
            bi0 = jnp.full((16,), KMAX, jnp.int32)
            bd, bi = lax.fori_loop(0, nch, ch_body, (bd0, bi0))

            d2min = jnp.min(bd)
            merge = (ncent > 0) & (d2min < MARGIN2)
            eqm = bd == d2min
            def _fast():
                ffs = plsc.all_reduce_ffs(eqm)
                return _vbcast(bi, jnp.minimum(ffs, 15))

            def _slow():
                return jnp.full((16,), jnp.min(jnp.where(eqm, bi, KMAX)),
                                jnp.int32)
            idxv = lax.cond(nch <= 1, _fast, _slow)

            tv = jnp.where(merge, idxv, jnp.full((16,), ncent, jnp.int32))
            tv = jnp.minimum(tv, CCAP - 1)
            cntf = plsc.load_gather(counts, [tv]).astype(jnp.float32)
            cidx = lanes * CCAP + tv
            cold = plsc.load_gather(centers, [cidx], mask=m_lo8)
            cnew = (cold * cntf + ev) / (cntf + 1.0)
            plsc.store_scatter(centers, [cidx], cnew, mask=m_lo8)
            plsc.addupdate_scatter(counts, [tv], ones_i, mask=m_l0)
            plsc.store_scatter(cids, [iv], tv, mask=m_l0)
            return ncent + jnp.where(merge, 0, 1)

        lax.fori_loop(0, n, point_body, jnp.int32(0))

        def slot_body(c, s):
            cv = counts[pl.ds(c * 16, 16)]
            el = cv >= MIN_CLUSTER
            eli = el.astype(jnp.int32)
            incl = plsc.cumsum(eli)
            slotv = jnp.where(el, s + (incl - eli), -1)
            slotv = jnp.where(slotv < SLOTS, slotv, -1)
            slotmap[pl.ds(c * 16, 16)] = slotv
            return s + jnp.sum(eli)
        lax.fori_loop(0, CCAP // 16, slot_body, jnp.int32(0))

        @plsc.parallel_loop(0, SLOTS * H // 16, unroll=4)
        def _(i):
            scnt[pl.ds(i * 16, 16)] = zi16
            ssx[pl.ds(i * 16, 16)] = zf16
            ssz[pl.ds(i * 16, 16)] = zf16

        @plsc.parallel_loop(0, MAX_LANES * H // 16, unroll=4)
        def _(i):
            obx[pl.ds(i * 16, 16)] = zf16
            oby[pl.ds(i * 16, 16)] = zf16
            obz[pl.ds(i * 16, 16)] = zf16
        cntv[pl.ds(0, 16)] = zi16

        def st_body(c, _):
            base = c * 16
            pidx = aidx[pl.ds(base, 16)]
            cid = cids[pl.ds(base, 16)]
            slot = plsc.load_gather(slotmap, [cid])
            valid = (slot >= 0) & ((lanes + base) < n)
            rows = pidx >> 8
            sidx = jnp.where(valid, slot * H + rows, 0)
            xv = xadj[pl.ds(base, 16)]
            zv = zact[pl.ds(base, 16)]
            for j in range(16):
                mj = (lanes == j) & valid
                plsc.addupdate_scatter(scnt, [sidx], ones_i, mask=mj)
                plsc.addupdate_scatter(ssx, [sidx], xv, mask=mj)
                plsc.addupdate_scatter(ssz, [sidx], zv, mask=mj)
            return 0
        lax.fori_loop(0, nchp, st_body, 0)

        def lane_body(s, lane):
            def nr_body(c9, acc):
                cv = scnt[pl.ds(s * H + c9 * 16, 16)]
                return acc + jnp.sum((cv > 0).astype(jnp.int32))
            nr = lax.fori_loop(0, H // 16, nr_body, jnp.int32(0))
            cand = nr >= 2
            do_lane = cand & (lane < MAX_LANES)

            @pl.when(do_lane)
            def _():
                def row_body(t, pos_carry):
                    c9 = (H // 16 - 1) - t
                    o = s * H + c9 * 16
                    cv = scnt[pl.ds(o, 16)]
                    pres = cv > 0
                    pi = pres.astype(jnp.int32)
                    denom = jnp.where(pres, cv, 1).astype(jnp.float32)
                    mean_x = ssx[pl.ds(o, 16)] / denom
                    mean_z = ssz[pl.ds(o, 16)] / denom
                    rr = (lanes + c9 * 16).astype(jnp.float32)
                    xvv = (100.0 - (rr + 0.5)) * 0.5
                    yvv = 64.0 - 0.5 * mean_x
                    incl = plsc.cumsum(lax.rev(pi, (0,)))
                    pos_local = lax.rev(incl, (0,)) - pi
                    pos = pos_local + pos_carry + lane * H
                    plsc.store_scatter(obx, [pos], xvv, mask=pres)
                    plsc.store_scatter(oby, [pos], yvv, mask=pres)
                    plsc.store_scatter(obz, [pos], mean_z, mask=pres)
                    return pos_carry + jnp.sum(pi)
                n_l = lax.fori_loop(0, H // 16, row_body, jnp.int32(0))
                plsc.store_scatter(
                    cntv, [jnp.full((16,), lane, jnp.int32)],
                    jnp.full((16,), n_l, jnp.int32), mask=lanes == 0)

            return lane + jnp.where(cand, 1, 0)
        lax.fori_loop(0, SLOTS, lane_body, jnp.int32(0))

        pltpu.sync_copy(obx, ox_out.at[b])
        pltpu.sync_copy(oby, oy_out.at[b])
        pltpu.sync_copy(obz, oz_out.at[b])
        pltpu.sync_copy(cntv, cnt_out.at[b])


def kernel(seg, embedding, offset_pred, z_pred, intrinsic, extrinsic):
    segf = seg.reshape(B, KMAX)
    embf = embedding.reshape(B * ND, KMAX)
    offf = offset_pred.reshape(B, KMAX)
    zf = z_pred.reshape(B, KMAX)

    ox, oy, oz, cnt = _fused_kernel(segf, embf, offf, zf)

    out = jnp.stack(
        [ox.reshape(B, MAX_LANES, H),
         oy.reshape(B, MAX_LANES, H),
         oz.reshape(B, MAX_LANES, H)], axis=-1)
    return out, cnt[:, :MAX_LANES]
